# initial kernel scaffold (unmeasured)
import jax
import jax.numpy as jnp
from jax import lax
from jax.experimental import pallas as pl
from jax.experimental.pallas import tpu as pltpu


def kernel(x, W):
    T, D = x.shape
    V = W.shape[1]
    N_CHUNKS = 8
    CH = V // N_CHUNKS

    xb = x.astype(jnp.bfloat16)
    Wb = W.astype(jnp.bfloat16)
    logits = jnp.dot(xb, Wb, preferred_element_type=jnp.float32)
    lb = logits.astype(jnp.bfloat16)

    def body(lb_ref, out_ref, peer_ref, stage_ref, send_sem, recv_sem, out_sem):
        my_x = lax.axis_index("x")
        my_y = lax.axis_index("y")
        my_z = lax.axis_index("z")
        peer = (1 - my_x, my_y, my_z)

        barrier = pltpu.get_barrier_semaphore()
        pl.semaphore_signal(
            barrier, 1, device_id=peer, device_id_type=pl.DeviceIdType.MESH
        )
        pl.semaphore_wait(barrier, 1)

        rdma = pltpu.make_async_remote_copy(
            src_ref=lb_ref,
            dst_ref=peer_ref,
            send_sem=send_sem,
            recv_sem=recv_sem,
            device_id=peer,
            device_id_type=pl.DeviceIdType.MESH,
        )
        rdma.start()
        rdma.wait()

        m = jnp.full((T, 1), -jnp.inf, dtype=jnp.float32)
        for ref in (lb_ref, peer_ref):
            for k in range(N_CHUNKS):
                chunk = ref[:, k * CH:(k + 1) * CH].astype(jnp.float32)
                m = jnp.maximum(m, jnp.max(chunk, axis=1, keepdims=True))
        s = jnp.zeros((T, 1), jnp.float32)
        for ref in (lb_ref, peer_ref):
            for k in range(N_CHUNKS):
                chunk = ref[:, k * CH:(k + 1) * CH].astype(jnp.float32)
                s = s + jnp.sum(jnp.exp(chunk - m), axis=1, keepdims=True)
        inv = 1.0 / s

        my_base = my_x * V
        peer_base = (1 - my_x) * V
        for ref, base in ((lb_ref, my_base), (peer_ref, peer_base)):
            for k in range(N_CHUNKS):
                slot = k % 2
                chunk = ref[:, k * CH:(k + 1) * CH].astype(jnp.float32)
                stage_ref[slot] = jnp.exp(chunk - m) * inv
                copy = pltpu.make_async_copy(
                    stage_ref.at[slot],
                    out_ref.at[:, pl.ds(base + k * CH, CH)],
                    out_sem.at[slot],
                )
                copy.start()
                copy.wait()

    return pl.pallas_call(
        body,
        out_shape=jax.ShapeDtypeStruct((T, 2 * V), jnp.float32),
        in_specs=[pl.BlockSpec(memory_space=pltpu.VMEM)],
        out_specs=pl.BlockSpec(memory_space=pltpu.ANY),
        scratch_shapes=[
            pltpu.VMEM((T, V), jnp.bfloat16),
            pltpu.VMEM((2, T, CH), jnp.float32),
            pltpu.SemaphoreType.DMA,
            pltpu.SemaphoreType.DMA,
            pltpu.SemaphoreType.DMA((2,)),
        ],
        compiler_params=pltpu.CompilerParams(collective_id=0),
    )(lb)


# baseline (device time: 823783 ns/iter reference)
import jax
import jax.numpy as jnp
from jax import lax
from jax.experimental import pallas as pl
from jax.experimental.pallas import tpu as pltpu


def kernel(x, W):
    T, D = x.shape
    V = W.shape[1]
    N_CHUNKS = 8
    CH = V // N_CHUNKS

    xb = x.astype(jnp.bfloat16)
    Wb = W.astype(jnp.bfloat16)
    logits = jnp.dot(xb, Wb, preferred_element_type=jnp.float32)
    lb = logits.astype(jnp.bfloat16)

    def body(lb_ref, out_ref, peer_ref, in_stage, out_stage,
             send_sem, recv_sem, in_sem, out_sem):
        my_x = lax.axis_index("x")
        my_y = lax.axis_index("y")
        my_z = lax.axis_index("z")
        peer = (1 - my_x, my_y, my_z)

        barrier = pltpu.get_barrier_semaphore()
        pl.semaphore_signal(
            barrier, 1, device_id=peer, device_id_type=pl.DeviceIdType.MESH
        )
        pl.semaphore_wait(barrier, 1)

        rdma = pltpu.make_async_remote_copy(
            src_ref=lb_ref,
            dst_ref=peer_ref,
            send_sem=send_sem,
            recv_sem=recv_sem,
            device_id=peer,
            device_id_type=pl.DeviceIdType.MESH,
        )
        rdma.start()
        rdma.wait()

        def load(ref, k):
            cp = pltpu.make_async_copy(
                ref.at[:, pl.ds(k * CH, CH)], in_stage, in_sem
            )
            cp.start()
            cp.wait()
            return in_stage[...].astype(jnp.float32)

        def stats_half(ref, carry):
            def step(k, carry):
                m, s = carry
                chunk = load(ref, k)
                m_new = jnp.maximum(
                    m, jnp.max(chunk, axis=1, keepdims=True)
                )
                s = s * jnp.exp(m - m_new) + jnp.sum(
                    jnp.exp(chunk - m_new), axis=1, keepdims=True
                )
                return m_new, s
            return lax.fori_loop(0, N_CHUNKS, step, carry)

        m0 = jnp.full((T, 1), -1e30, dtype=jnp.float32)
        s0 = jnp.zeros((T, 1), jnp.float32)
        m, s = stats_half(peer_ref, stats_half(lb_ref, (m0, s0)))
        inv = 1.0 / s

        def norm_half(ref, base):
            def step(k, _):
                chunk = load(ref, k)
                out_stage[...] = jnp.exp(chunk - m) * inv
                cp = pltpu.make_async_copy(
                    out_stage,
                    out_ref.at[:, pl.ds(base + k * CH, CH)],
                    out_sem,
                )
                cp.start()
                cp.wait()
                return 0
            lax.fori_loop(0, N_CHUNKS, step, 0)

        norm_half(lb_ref, my_x * V)
        norm_half(peer_ref, (1 - my_x) * V)

    out, _ = pl.pallas_call(
        body,
        out_shape=[
            jax.ShapeDtypeStruct((T, 2 * V), jnp.float32),
            jax.ShapeDtypeStruct((T, V), jnp.bfloat16),
        ],
        in_specs=[pl.BlockSpec(memory_space=pl.ANY)],
        out_specs=[
            pl.BlockSpec(memory_space=pl.ANY),
            pl.BlockSpec(memory_space=pl.ANY),
        ],
        scratch_shapes=[
            pltpu.VMEM((T, CH), jnp.bfloat16),
            pltpu.VMEM((T, CH), jnp.float32),
            pltpu.SemaphoreType.DMA,
            pltpu.SemaphoreType.DMA,
            pltpu.SemaphoreType.DMA,
            pltpu.SemaphoreType.DMA,
        ],
        compiler_params=pltpu.CompilerParams(collective_id=0),
    )(lb)
    return out


# device time: 510743 ns/iter; 1.6129x vs baseline; 1.6129x over previous
import jax
import jax.numpy as jnp
from jax import lax
from jax.experimental import pallas as pl
from jax.experimental.pallas import tpu as pltpu


def kernel(x, W):
    T, D = x.shape
    V = W.shape[1]
    NC = 16
    CH = V // NC
    HOLD = 8
    XC = D // 512

    def body(x_ref, W_ref, out_ref, lb_hbm, peer_hbm,
             xb, x_st, Wf, Wb, lb_st, in_st, out_st, stats_s, stats_r,
             send_sems, recv_sems, stats_send_sem, stats_recv_sem,
             x_sems, w_sems, lb_sem, in_sem, out_sem):
        my_x = lax.axis_index("x")
        my_y = lax.axis_index("y")
        my_z = lax.axis_index("z")
        peer = (1 - my_x, my_y, my_z)

        barrier = pltpu.get_barrier_semaphore()
        pl.semaphore_signal(
            barrier, 1, device_id=peer, device_id_type=pl.DeviceIdType.MESH
        )
        pl.semaphore_wait(barrier, 1)

        def x_load(j, slot):
            return pltpu.make_async_copy(
                x_ref.at[:, pl.ds(j * 512, 512)], x_st.at[slot], x_sems.at[slot]
            )

        x_load(0, 0).start()

        def xstep(j, _):
            slot = lax.rem(j, 2)
            x_load(j, slot).wait()

            @pl.when(j < XC - 1)
            def _prefetch():
                x_load(j + 1, lax.rem(j + 1, 2)).start()

            xb[:, pl.ds(j * 512, 512)] = x_st[slot].astype(jnp.bfloat16)
            return 0

        lax.fori_loop(0, XC, xstep, 0)

        def chunk_rdma(k):
            return pltpu.make_async_remote_copy(
                src_ref=lb_hbm.at[:, pl.ds(k * CH, CH)],
                dst_ref=peer_hbm.at[:, pl.ds(k * CH, CH)],
                send_sem=send_sems.at[k],
                recv_sem=recv_sems.at[k],
                device_id=peer,
                device_id_type=pl.DeviceIdType.MESH,
            )

        def w_load(k, slot):
            return pltpu.make_async_copy(
                W_ref.at[:, pl.ds(k * CH, CH)], Wf.at[slot], w_sems.at[slot]
            )

        w_load(0, 0).start()

        def gstep(k, carry):
            m, s = carry
            slot = lax.rem(k, 2)
            w_load(k, slot).wait()

            @pl.when(k < NC - 1)
            def _prefetch():
                w_load(k + 1, lax.rem(k + 1, 2)).start()

            Wb[...] = Wf[slot].astype(jnp.bfloat16)
            logits = jnp.dot(
                xb[...], Wb[...], preferred_element_type=jnp.float32
            )
            m_new = jnp.maximum(m, jnp.max(logits, axis=1, keepdims=True))
            s = s * jnp.exp(m - m_new) + jnp.sum(
                jnp.exp(logits - m_new), axis=1, keepdims=True
            )
            lb_st[...] = logits.astype(jnp.bfloat16)
            st = pltpu.make_async_copy(
                lb_st, lb_hbm.at[:, pl.ds(k * CH, CH)], lb_sem
            )
            st.start()
            st.wait()

            @pl.when(k < HOLD)
            def _send():
                chunk_rdma(k).start()

            return m_new, s

        m0 = jnp.full((T, 1), -1e30, dtype=jnp.float32)
        s0 = jnp.zeros((T, 1), jnp.float32)
        m, s = lax.fori_loop(0, NC, gstep, (m0, s0))

        stats_s[0, :, :] = jnp.broadcast_to(m, (T, 128))
        stats_s[1, :, :] = jnp.broadcast_to(s, (T, 128))
        srdma = pltpu.make_async_remote_copy(
            src_ref=stats_s,
            dst_ref=stats_r,
            send_sem=stats_send_sem,
            recv_sem=stats_recv_sem,
            device_id=peer,
            device_id_type=pl.DeviceIdType.MESH,
        )
        srdma.start()

        def rel(k, _):
            chunk_rdma(k).start()
            return 0

        lax.fori_loop(HOLD, NC, rel, 0)

        srdma.wait_recv()
        m_p = stats_r[0, :, 0:1]
        s_p = stats_r[1, :, 0:1]
        M = jnp.maximum(m, m_p)
        inv = 1.0 / (s * jnp.exp(m - M) + s_p * jnp.exp(m_p - M))

        my_base = my_x * V
        peer_base = (1 - my_x) * V

        def norm(ref_hbm, base, k):
            ld = pltpu.make_async_copy(
                ref_hbm.at[:, pl.ds(k * CH, CH)], in_st, in_sem
            )
            ld.start()
            ld.wait()
            out_st[...] = jnp.exp(in_st[...].astype(jnp.float32) - M) * inv
            st = pltpu.make_async_copy(
                out_st, out_ref.at[:, pl.ds(base + k * CH, CH)], out_sem
            )
            st.start()
            st.wait()

        def mstep(k, _):
            norm(lb_hbm, my_base, k)
            return 0

        lax.fori_loop(0, NC, mstep, 0)

        def pstep(k, _):
            chunk_rdma(k).wait_recv()
            norm(peer_hbm, peer_base, k)
            return 0

        lax.fori_loop(0, NC, pstep, 0)

        def dstep(k, _):
            chunk_rdma(k).wait_send()
            return 0

        lax.fori_loop(0, NC, dstep, 0)
        srdma.wait_send()

    out, _, _ = pl.pallas_call(
        body,
        out_shape=[
            jax.ShapeDtypeStruct((T, 2 * V), jnp.float32),
            jax.ShapeDtypeStruct((T, V), jnp.bfloat16),
            jax.ShapeDtypeStruct((T, V), jnp.bfloat16),
        ],
        in_specs=[
            pl.BlockSpec(memory_space=pl.ANY),
            pl.BlockSpec(memory_space=pl.ANY),
        ],
        out_specs=[
            pl.BlockSpec(memory_space=pl.ANY),
            pl.BlockSpec(memory_space=pl.ANY),
            pl.BlockSpec(memory_space=pl.ANY),
        ],
        scratch_shapes=[
            pltpu.VMEM((T, D), jnp.bfloat16),
            pltpu.VMEM((2, T, 512), jnp.float32),
            pltpu.VMEM((2, D, CH), jnp.float32),
            pltpu.VMEM((D, CH), jnp.bfloat16),
            pltpu.VMEM((T, CH), jnp.bfloat16),
            pltpu.VMEM((T, CH), jnp.bfloat16),
            pltpu.VMEM((T, CH), jnp.float32),
            pltpu.VMEM((2, T, 128), jnp.float32),
            pltpu.VMEM((2, T, 128), jnp.float32),
            pltpu.SemaphoreType.DMA((NC,)),
            pltpu.SemaphoreType.DMA((NC,)),
            pltpu.SemaphoreType.DMA,
            pltpu.SemaphoreType.DMA,
            pltpu.SemaphoreType.DMA((2,)),
            pltpu.SemaphoreType.DMA((2,)),
            pltpu.SemaphoreType.DMA,
            pltpu.SemaphoreType.DMA,
            pltpu.SemaphoreType.DMA,
        ],
        compiler_params=pltpu.CompilerParams(
            collective_id=0, vmem_limit_bytes=63 * 1024 * 1024
        ),
    )(x, W)
    return out
